# Initial kernel scaffold; baseline (speedup 1.0000x reference)
import jax
import jax.numpy as jnp
from jax.experimental import pallas as pl


def kernel(x_ncl, w2, b2, w3, b3, w4, b4, fc_w, fc_b):
    raise NotImplementedError("write your pallas kernel here")


# speedup vs baseline: 1.2682x; 1.2682x over previous
import jax
import jax.numpy as jnp
from jax.experimental import pallas as pl
from jax.experimental.pallas import tpu as pltpu

_NT = 4


def _body(bb, L, C, O):
    F = 3 * O

    def body(x_ref, w_ref, b_ref, fcw_ref, fcb_ref, out_ref):
        feats = []
        for b in range(bb):
            xb = x_ref[b].astype(jnp.bfloat16)
            parts = [xb]
            for j in range(1, _NT):
                parts.append(jnp.concatenate([xb[:, j:], xb[:, :j]], axis=1))
            xcat = jnp.concatenate(parts, axis=0)
            y = jax.lax.dot_general(
                xcat, w_ref[...],
                dimension_numbers=(((0,), (0,)), ((), ())),
                preferred_element_type=jnp.float32)
            y = jnp.maximum(y + b_ref[...], 0.0)

            body_max = jnp.max(y[:L - 8], axis=0, keepdims=True)
            tail = y[L - 8:]
            t_idx = (L - 8) + jax.lax.broadcasted_iota(jnp.int32, (8, F), 0)
            blk = jax.lax.broadcasted_iota(jnp.int32, (8, F), 1) // O
            tail = jnp.where(t_idx < (L - 1 - blk), tail, 0.0)
            tail_max = jnp.max(tail, axis=0, keepdims=True)
            feats.append(jnp.maximum(body_max, tail_max))

        feat = jnp.concatenate(feats, axis=0)
        logits = jnp.dot(feat, fcw_ref[...],
                         preferred_element_type=jnp.float32) + fcb_ref[...]
        m = jnp.max(logits, axis=1, keepdims=True)
        e = jnp.exp(logits - m)
        s = jnp.sum(e, axis=1, keepdims=True)
        out_ref[...] = e * pl.reciprocal(s, approx=True)

    return body


def kernel(x_ncl, w2, b2, w3, b3, w4, b4, fc_w, fc_b):
    B, C, L = x_ncl.shape
    O = w2.shape[0]
    F = 3 * O

    zeros = jnp.zeros((C, O), jnp.float32)

    def tap(j):
        cols = [jnp.transpose(w[:, :, j], (1, 0)) if j < k else zeros
                for w, k in ((w2, 2), (w3, 3), (w4, 4))]
        return jnp.concatenate(cols, axis=1)

    w_cat = jnp.concatenate([tap(j) for j in range(_NT)],
                            axis=0).astype(jnp.bfloat16)
    bias = jnp.concatenate([b2, b3, b4]).reshape(1, F)
    fcw = jnp.transpose(fc_w, (1, 0))
    fcb = fc_b.reshape(1, 2)

    bb = 8 if (B % 8 == 0) else B
    grid = (B // bb,)

    return pl.pallas_call(
        _body(bb, L, C, O),
        out_shape=jax.ShapeDtypeStruct((B, 2), jnp.float32),
        grid=grid,
        in_specs=[
            pl.BlockSpec((bb, C, L), lambda i: (i, 0, 0)),
            pl.BlockSpec((_NT * C, F), lambda i: (0, 0)),
            pl.BlockSpec((1, F), lambda i: (0, 0)),
            pl.BlockSpec((F, 2), lambda i: (0, 0)),
            pl.BlockSpec((1, 2), lambda i: (0, 0)),
        ],
        out_specs=pl.BlockSpec((bb, 2), lambda i: (i, 0)),
        compiler_params=pltpu.CompilerParams(
            dimension_semantics=("parallel",)),
    )(x_ncl, w_cat, bias, fcw, fcb)


# speedup vs baseline: 1.4426x; 1.1375x over previous
import jax
import jax.numpy as jnp
from jax.experimental import pallas as pl
from jax.experimental.pallas import tpu as pltpu

_NT = 4


def _body(bb, L, C, O):
    F = 3 * O

    def body(x_ref, w_ref, b_ref, fcw_ref, fcb_ref, out_ref):
        cols = []
        for b in range(bb):
            xb = x_ref[b].astype(jnp.bfloat16)
            parts = [xb]
            for j in range(1, _NT):
                parts.append(jnp.concatenate([xb[:, j:], xb[:, :j]], axis=1))
            cols.append(jnp.concatenate(parts, axis=0))
        xcat = jnp.concatenate(cols, axis=1)
        y = jax.lax.dot_general(
            xcat, w_ref[...],
            dimension_numbers=(((0,), (0,)), ((), ())),
            preferred_element_type=jnp.float32)

        t_idx = (L - 8) + jax.lax.broadcasted_iota(jnp.int32, (8, F), 0)
        blk = jax.lax.broadcasted_iota(jnp.int32, (8, F), 1) // O
        valid = t_idx < (L - 1 - blk)
        feats = []
        for b in range(bb):
            yb = y[b * L:(b + 1) * L]
            body_max = jnp.max(yb[:L - 8], axis=0, keepdims=True)
            tail = jnp.where(valid, yb[L - 8:], -jnp.inf)
            tail_max = jnp.max(tail, axis=0, keepdims=True)
            feats.append(jnp.maximum(body_max, tail_max))

        feat = jnp.maximum(jnp.concatenate(feats, axis=0) + b_ref[...], 0.0)
        logits = jnp.dot(feat, fcw_ref[...],
                         preferred_element_type=jnp.float32) + fcb_ref[...]
        m = jnp.max(logits, axis=1, keepdims=True)
        e = jnp.exp(logits - m)
        s = jnp.sum(e, axis=1, keepdims=True)
        out_ref[...] = e * pl.reciprocal(s, approx=True)

    return body


def kernel(x_ncl, w2, b2, w3, b3, w4, b4, fc_w, fc_b):
    B, C, L = x_ncl.shape
    O = w2.shape[0]
    F = 3 * O

    zeros = jnp.zeros((C, O), jnp.float32)

    def tap(j):
        cols = [jnp.transpose(w[:, :, j], (1, 0)) if j < k else zeros
                for w, k in ((w2, 2), (w3, 3), (w4, 4))]
        return jnp.concatenate(cols, axis=1)

    w_cat = jnp.concatenate([tap(j) for j in range(_NT)],
                            axis=0).astype(jnp.bfloat16)
    bias = jnp.concatenate([b2, b3, b4]).reshape(1, F)
    fcw = jnp.transpose(fc_w, (1, 0))
    fcb = fc_b.reshape(1, 2)

    bb = 8 if (B % 8 == 0) else B
    grid = (B // bb,)

    return pl.pallas_call(
        _body(bb, L, C, O),
        out_shape=jax.ShapeDtypeStruct((B, 2), jnp.float32),
        grid=grid,
        in_specs=[
            pl.BlockSpec((bb, C, L), lambda i: (i, 0, 0)),
            pl.BlockSpec((_NT * C, F), lambda i: (0, 0)),
            pl.BlockSpec((1, F), lambda i: (0, 0)),
            pl.BlockSpec((F, 2), lambda i: (0, 0)),
            pl.BlockSpec((1, 2), lambda i: (0, 0)),
        ],
        out_specs=pl.BlockSpec((bb, 2), lambda i: (i, 0)),
        compiler_params=pltpu.CompilerParams(
            dimension_semantics=("parallel",)),
    )(x_ncl, w_cat, bias, fcw, fcb)


# speedup vs baseline: 1.5589x; 1.0806x over previous
import jax
import jax.numpy as jnp
from jax.experimental import pallas as pl
from jax.experimental.pallas import tpu as pltpu

_NT = 4


def _body(bb, L, C, O):
    F = 3 * O

    def body(x_ref, w_ref, b_ref, fcw_ref, fcb_ref, out_ref):
        cols = []
        for b in range(bb):
            xb = x_ref[b].astype(jnp.bfloat16)
            parts = [xb]
            for j in range(1, _NT):
                parts.append(jnp.concatenate([xb[:, j:], xb[:, :j]], axis=1))
            cols.append(jnp.concatenate(parts, axis=0))
        xcat = jnp.concatenate(cols, axis=1)
        y = jax.lax.dot_general(
            xcat, w_ref[...],
            dimension_numbers=(((0,), (0,)), ((), ())),
            preferred_element_type=jnp.float32)

        t_idx = (L - 8) + jax.lax.broadcasted_iota(jnp.int32, (8, F), 0)
        blk = jax.lax.broadcasted_iota(jnp.int32, (8, F), 1) // O
        valid = t_idx < (L - 1 - blk)
        feats = []
        for b in range(bb):
            yb = y[b * L:(b + 1) * L]
            body_max = jnp.max(yb[:L - 8], axis=0, keepdims=True)
            tail = jnp.where(valid, yb[L - 8:], -jnp.inf)
            tail_max = jnp.max(tail, axis=0, keepdims=True)
            feats.append(jnp.maximum(body_max, tail_max))

        feat = jnp.maximum(jnp.concatenate(feats, axis=0) + b_ref[...], 0.0)
        logits = jnp.dot(feat, fcw_ref[...],
                         preferred_element_type=jnp.float32) + fcb_ref[...]
        m = jnp.max(logits, axis=1, keepdims=True)
        e = jnp.exp(logits - m)
        s = jnp.sum(e, axis=1, keepdims=True)
        out_ref[...] = e * pl.reciprocal(s, approx=True)

    return body


def kernel(x_ncl, w2, b2, w3, b3, w4, b4, fc_w, fc_b):
    B, C, L = x_ncl.shape
    O = w2.shape[0]
    F = 3 * O

    zeros = jnp.zeros((C, O), jnp.float32)

    def tap(j):
        cols = [jnp.transpose(w[:, :, j], (1, 0)) if j < k else zeros
                for w, k in ((w2, 2), (w3, 3), (w4, 4))]
        return jnp.concatenate(cols, axis=1)

    w_cat = jnp.concatenate([tap(j) for j in range(_NT)],
                            axis=0).astype(jnp.bfloat16)
    bias = jnp.concatenate([b2, b3, b4]).reshape(1, F)
    fcw = jnp.transpose(fc_w, (1, 0))
    fcb = fc_b.reshape(1, 2)

    bb = 16 if (B % 16 == 0) else (8 if (B % 8 == 0) else B)
    grid = (B // bb,)

    return pl.pallas_call(
        _body(bb, L, C, O),
        out_shape=jax.ShapeDtypeStruct((B, 2), jnp.float32),
        grid=grid,
        in_specs=[
            pl.BlockSpec((bb, C, L), lambda i: (i, 0, 0)),
            pl.BlockSpec((_NT * C, F), lambda i: (0, 0)),
            pl.BlockSpec((1, F), lambda i: (0, 0)),
            pl.BlockSpec((F, 2), lambda i: (0, 0)),
            pl.BlockSpec((1, 2), lambda i: (0, 0)),
        ],
        out_specs=pl.BlockSpec((bb, 2), lambda i: (i, 0)),
        compiler_params=pltpu.CompilerParams(
            dimension_semantics=("parallel",)),
    )(x_ncl, w_cat, bias, fcw, fcb)


# speedup vs baseline: 1.6364x; 1.0497x over previous
import jax
import jax.numpy as jnp
from jax.experimental import pallas as pl
from jax.experimental.pallas import tpu as pltpu

_NT = 4


def _body(bb, L, C, O):
    F = 3 * O

    def body(x_ref, w_ref, b_ref, fcw_ref, fcb_ref, out_ref):
        cols = []
        for b in range(bb):
            xb = x_ref[b].astype(jnp.bfloat16)
            parts = [xb]
            for j in range(1, _NT):
                parts.append(jnp.concatenate([xb[:, j:], xb[:, :j]], axis=1))
            cols.append(jnp.concatenate(parts, axis=0))
        xcat = jnp.concatenate(cols, axis=1)
        y = jax.lax.dot_general(
            xcat, w_ref[...],
            dimension_numbers=(((0,), (0,)), ((), ())),
            preferred_element_type=jnp.float32)

        t_idx = (L - 8) + jax.lax.broadcasted_iota(jnp.int32, (8, F), 0)
        blk = jax.lax.broadcasted_iota(jnp.int32, (8, F), 1) // O
        valid = t_idx < (L - 1 - blk)
        feats = []
        for b in range(bb):
            yb = y[b * L:(b + 1) * L]
            body_max = jnp.max(yb[:L - 8], axis=0, keepdims=True)
            tail = jnp.where(valid, yb[L - 8:], -jnp.inf)
            tail_max = jnp.max(tail, axis=0, keepdims=True)
            feats.append(jnp.maximum(body_max, tail_max))

        feat = jnp.maximum(jnp.concatenate(feats, axis=0) + b_ref[...], 0.0)
        logits = jnp.dot(feat, fcw_ref[...],
                         preferred_element_type=jnp.float32) + fcb_ref[...]
        m = jnp.max(logits, axis=1, keepdims=True)
        e = jnp.exp(logits - m)
        s = jnp.sum(e, axis=1, keepdims=True)
        out_ref[...] = e * pl.reciprocal(s, approx=True)

    return body


def kernel(x_ncl, w2, b2, w3, b3, w4, b4, fc_w, fc_b):
    B, C, L = x_ncl.shape
    O = w2.shape[0]
    F = 3 * O

    zeros = jnp.zeros((C, O), jnp.float32)

    def tap(j):
        cols = [jnp.transpose(w[:, :, j], (1, 0)) if j < k else zeros
                for w, k in ((w2, 2), (w3, 3), (w4, 4))]
        return jnp.concatenate(cols, axis=1)

    w_cat = jnp.concatenate([tap(j) for j in range(_NT)],
                            axis=0).astype(jnp.bfloat16)
    bias = jnp.concatenate([b2, b3, b4]).reshape(1, F)
    fcw = jnp.transpose(fc_w, (1, 0))
    fcb = fc_b.reshape(1, 2)

    bb = next((c for c in (32, 16, 8) if B % c == 0), B)
    grid = (B // bb,)

    return pl.pallas_call(
        _body(bb, L, C, O),
        out_shape=jax.ShapeDtypeStruct((B, 2), jnp.float32),
        grid=grid,
        in_specs=[
            pl.BlockSpec((bb, C, L), lambda i: (i, 0, 0)),
            pl.BlockSpec((_NT * C, F), lambda i: (0, 0)),
            pl.BlockSpec((1, F), lambda i: (0, 0)),
            pl.BlockSpec((F, 2), lambda i: (0, 0)),
            pl.BlockSpec((1, 2), lambda i: (0, 0)),
        ],
        out_specs=pl.BlockSpec((bb, 2), lambda i: (i, 0)),
        compiler_params=pltpu.CompilerParams(
            dimension_semantics=("parallel",)),
    )(x_ncl, w_cat, bias, fcw, fcb)
